# baseline (speedup 1.0000x reference)
import jax
import jax.numpy as jnp
from jax import lax
from jax.experimental import pallas as pl
from jax.experimental.pallas import tpu as pltpu
from jax.experimental.pallas import tpu_sc as plsc

NC, NS = 2, 16
NW = NC * NS


def _project(x, W1):
    N, D = x.shape
    H = W1.shape[1]
    BN = 1000

    def body(x_ref, w_ref, o_ref):
        o_ref[...] = jnp.dot(x_ref[...], w_ref[...],
                             preferred_element_type=jnp.float32)

    return pl.pallas_call(
        body,
        grid=(N // BN,),
        in_specs=[pl.BlockSpec((BN, D), lambda i: (i, 0)),
                  pl.BlockSpec((D, H), lambda i: (0, 0))],
        out_specs=pl.BlockSpec((BN, H), lambda i: (i, 0)),
        out_shape=jax.ShapeDtypeStruct((N, H), jnp.float32),
    )(x, W1)


def _sc_aggregate(ei4, y, z_agg, z_deg, ones):
    NP = z_agg.shape[0]
    H = y.shape[1]
    _, _, NB, B = ei4.shape
    DW = z_deg.shape[1]
    RPT = NP // NS
    mesh = plsc.VectorSubcoreMesh(core_axis_name="c", subcore_axis_name="s",
                                  num_cores=NC, num_subcores=NS)

    def body(ei_hbm, y_hbm, za_hbm, zd_hbm, ones_hbm,
             agg_hbm, deg_hbm,
             src_v, dst_v, rows0_v, rows1_v, rows2_v, rows3_v, ones_v,
             agg_sh, deg_sh,
             gsem0, gsem1, gsem2, gsem3, ssem0, ssem1, ssem2, ssem3, osem):
        cid = lax.axis_index("c")
        sid = lax.axis_index("s")
        wid = cid * NS + sid
        r0 = sid * RPT
        pltpu.sync_copy(za_hbm.at[pl.ds(r0, RPT)], agg_sh.at[pl.ds(r0, RPT)])
        pltpu.sync_copy(zd_hbm.at[pl.ds(r0, RPT)], deg_sh.at[pl.ds(r0, RPT)])
        pltpu.sync_copy(ei_hbm.at[0, wid], src_v)
        pltpu.sync_copy(ei_hbm.at[1, wid], dst_v)
        pltpu.sync_copy(ones_hbm, ones_v)
        plsc.subcore_barrier()

        rows = (rows0_v, rows1_v, rows2_v, rows3_v)
        gsems = (gsem0, gsem1, gsem2, gsem3)
        ssems = (ssem0, ssem1, ssem2, ssem3)
        pltpu.async_copy(y_hbm.at[dst_v.at[0]], rows0_v, gsem0)
        pltpu.async_copy(y_hbm.at[dst_v.at[1]], rows1_v, gsem1)

        def half(j, u):
            b = u % 4
            b2 = (u + 2) % 4
            pltpu.make_async_copy(y_hbm.at[dst_v.at[j]], rows[b],
                                  gsems[b]).wait()
            pltpu.async_copy(rows[b], agg_sh.at[src_v.at[j]], ssems[b],
                             add=True)
            pltpu.async_copy(ones_v, deg_sh.at[src_v.at[j]], osem, add=True)

            @pl.when(j >= 2)
            def _():
                pltpu.make_async_copy(rows[b2], agg_sh.at[src_v.at[j]],
                                      ssems[b2]).wait()
                pltpu.make_async_copy(ones_v, deg_sh.at[src_v.at[j]],
                                      osem).wait()

            @pl.when(j + 2 < NB)
            def _():
                pltpu.async_copy(y_hbm.at[dst_v.at[j + 2]], rows[b2],
                                 gsems[b2])

        def step(i, carry):
            for u in range(4):
                half(4 * i + u, u)
            return carry

        lax.fori_loop(0, NB // 4, step, 0)
        pltpu.make_async_copy(rows[(NB - 2) % 4],
                              agg_sh.at[src_v.at[NB - 2]],
                              ssems[(NB - 2) % 4]).wait()
        pltpu.make_async_copy(rows[(NB - 1) % 4],
                              agg_sh.at[src_v.at[NB - 1]],
                              ssems[(NB - 1) % 4]).wait()
        pltpu.make_async_copy(ones_v, deg_sh.at[src_v.at[NB - 2]],
                              osem).wait()
        pltpu.make_async_copy(ones_v, deg_sh.at[src_v.at[NB - 1]],
                              osem).wait()
        plsc.subcore_barrier()
        pltpu.sync_copy(agg_sh.at[pl.ds(r0, RPT)],
                        agg_hbm.at[cid, pl.ds(r0, RPT)])
        pltpu.sync_copy(deg_sh.at[pl.ds(r0, RPT)],
                        deg_hbm.at[cid, pl.ds(r0, RPT)])

    f = pl.kernel(
        body,
        out_type=(jax.ShapeDtypeStruct((NC, NP, H), jnp.float32),
                  jax.ShapeDtypeStruct((NC, NP, DW), jnp.float32)),
        mesh=mesh,
        scratch_types=[
            pltpu.VMEM((NB, B), jnp.int32),
            pltpu.VMEM((NB, B), jnp.int32),
            pltpu.VMEM((B, H), jnp.float32),
            pltpu.VMEM((B, H), jnp.float32),
            pltpu.VMEM((B, H), jnp.float32),
            pltpu.VMEM((B, H), jnp.float32),
            pltpu.VMEM((B, DW), jnp.float32),
            pltpu.VMEM_SHARED((NP, H), jnp.float32),
            pltpu.VMEM_SHARED((NP, DW), jnp.float32),
            pltpu.SemaphoreType.DMA,
            pltpu.SemaphoreType.DMA,
            pltpu.SemaphoreType.DMA,
            pltpu.SemaphoreType.DMA,
            pltpu.SemaphoreType.DMA,
            pltpu.SemaphoreType.DMA,
            pltpu.SemaphoreType.DMA,
            pltpu.SemaphoreType.DMA,
            pltpu.SemaphoreType.DMA,
        ],
        compiler_params=pltpu.CompilerParams(use_tc_tiling_on_sc=False),
    )
    return f(ei4, y, z_agg, z_deg, ones)


def _epilogue(y, agg, deg, b1, W2, b2, W3, b3):
    N, H = y.shape
    BN = 2000
    G = N // BN

    def body(y_ref, agg_ref, deg_ref, b1_ref, W2_ref, b2_ref, W3_ref, b3_ref,
             o_ref, acc_ref):
        i = pl.program_id(0)

        @pl.when(i == 0)
        def _():
            acc_ref[...] = jnp.zeros_like(acc_ref)

        a = agg_ref[0] + agg_ref[1]
        d = jnp.maximum(deg_ref[0, :, 0:1] + deg_ref[1, :, 0:1], 1.0)
        h = jnp.maximum(y_ref[...] + a * (1.0 / d) + b1_ref[...], 0.0)
        h = jnp.maximum(
            jnp.dot(h, W2_ref[...], preferred_element_type=jnp.float32)
            + b2_ref[...], 0.0)
        acc_ref[...] += jnp.sum(h, axis=0, keepdims=True)

        @pl.when(i == G - 1)
        def _():
            m = acc_ref[...] / N
            o_ref[...] = jnp.tanh(
                jnp.dot(m, W3_ref[...], preferred_element_type=jnp.float32)
                + b3_ref[...])

    return pl.pallas_call(
        body,
        grid=(G,),
        in_specs=[
            pl.BlockSpec((BN, H), lambda i: (i, 0)),
            pl.BlockSpec((2, BN, H), lambda i: (0, i, 0)),
            pl.BlockSpec((2, BN, 16), lambda i: (0, i, 0)),
            pl.BlockSpec(b1.shape, lambda i: (0,)),
            pl.BlockSpec(W2.shape, lambda i: (0, 0)),
            pl.BlockSpec(b2.shape, lambda i: (0,)),
            pl.BlockSpec(W3.shape, lambda i: (0, 0)),
            pl.BlockSpec(b3.shape, lambda i: (0,)),
        ],
        out_specs=pl.BlockSpec((1, 1), lambda i: (0, 0)),
        out_shape=jax.ShapeDtypeStruct((1, 1), jnp.float32),
        scratch_shapes=[pltpu.VMEM((1, H), jnp.float32)],
    )(y, agg, deg, b1, W2, b2, W3, b3)


def kernel(x, edge_index, W1, b1, W2, b2, W3, b3):
    N, D = x.shape
    H = W1.shape[1]
    E = edge_index.shape[1]
    EPW = E // NW
    B = 125
    NB = EPW // B
    ei4 = edge_index.reshape(2, NW, NB, B)
    y = _project(x, W1)
    NP_ = ((N + 8 * NS - 1) // (8 * NS)) * (8 * NS)
    z_agg = jnp.zeros((NP_, H), jnp.float32)
    z_deg = jnp.zeros((NP_, 16), jnp.float32)
    ones = jnp.ones((B, 16), jnp.float32)
    agg, deg = _sc_aggregate(ei4, y, z_agg, z_deg, ones)
    out = _epilogue(y, agg, deg, b1, W2, b2, W3, b3)
    return out[0, 0]


# speedup vs baseline: 1.0302x; 1.0302x over previous
import jax
import jax.numpy as jnp
from jax import lax
from jax.experimental import pallas as pl
from jax.experimental.pallas import tpu as pltpu
from jax.experimental.pallas import tpu_sc as plsc

NC, NS = 2, 16
NW = NC * NS


def _project(x, W1):
    N, D = x.shape
    H = W1.shape[1]
    BN = 1000

    def body(x_ref, w_ref, o_ref):
        o_ref[...] = jnp.dot(x_ref[...], w_ref[...],
                             preferred_element_type=jnp.float32)

    return pl.pallas_call(
        body,
        grid=(N // BN,),
        in_specs=[pl.BlockSpec((BN, D), lambda i: (i, 0)),
                  pl.BlockSpec((D, H), lambda i: (0, 0))],
        out_specs=pl.BlockSpec((BN, H), lambda i: (i, 0)),
        out_shape=jax.ShapeDtypeStruct((N, H), jnp.float32),
    )(x, W1)


def _sc_aggregate(ei4, y, z_agg, z_deg, ones):
    NP = z_agg.shape[0]
    H = y.shape[1]
    _, _, NB, B = ei4.shape
    DW = z_deg.shape[1]
    RPT = NP // NS
    mesh = plsc.VectorSubcoreMesh(core_axis_name="c", subcore_axis_name="s",
                                  num_cores=NC, num_subcores=NS)

    def body(ei_hbm, y_hbm, za_hbm, zd_hbm, ones_hbm,
             agg_hbm, deg_hbm,
             src_v, dst_v, rows0_v, rows1_v, rows2_v, rows3_v, ones_v,
             agg_sh, deg_sh,
             gsem0, gsem1, gsem2, gsem3, ssem0, ssem1, ssem2, ssem3):
        cid = lax.axis_index("c")
        sid = lax.axis_index("s")
        wid = cid * NS + sid
        r0 = sid * RPT
        pltpu.sync_copy(za_hbm.at[pl.ds(r0, RPT)], agg_sh.at[pl.ds(r0, RPT)])
        pltpu.sync_copy(zd_hbm.at[pl.ds(r0, RPT)], deg_sh.at[pl.ds(r0, RPT)])
        pltpu.sync_copy(ei_hbm.at[0, wid], src_v)
        pltpu.sync_copy(ei_hbm.at[1, wid], dst_v)
        pltpu.sync_copy(ones_hbm, ones_v)
        plsc.subcore_barrier()

        rows = (rows0_v, rows1_v, rows2_v, rows3_v)
        gsems = (gsem0, gsem1, gsem2, gsem3)
        ssems = (ssem0, ssem1, ssem2, ssem3)
        pltpu.async_copy(y_hbm.at[dst_v.at[0]], rows0_v, gsem0)
        pltpu.async_copy(y_hbm.at[dst_v.at[1]], rows1_v, gsem1)

        def half(j, u):
            b = u % 4
            b2 = (u + 2) % 4
            pltpu.make_async_copy(y_hbm.at[dst_v.at[j]], rows[b],
                                  gsems[b]).wait()
            pltpu.async_copy(rows[b], agg_sh.at[src_v.at[j]], ssems[b],
                             add=True)
            pltpu.sync_copy(ones_v, deg_sh.at[src_v.at[j]], add=True)

            @pl.when(j >= 2)
            def _():
                pltpu.make_async_copy(rows[b2], agg_sh.at[src_v.at[j]],
                                      ssems[b2]).wait()

            @pl.when(j + 2 < NB)
            def _():
                pltpu.async_copy(y_hbm.at[dst_v.at[j + 2]], rows[b2],
                                 gsems[b2])

        def step(i, carry):
            for u in range(4):
                half(4 * i + u, u)
            return carry

        lax.fori_loop(0, NB // 4, step, 0)
        pltpu.make_async_copy(rows[(NB - 2) % 4],
                              agg_sh.at[src_v.at[NB - 2]],
                              ssems[(NB - 2) % 4]).wait()
        pltpu.make_async_copy(rows[(NB - 1) % 4],
                              agg_sh.at[src_v.at[NB - 1]],
                              ssems[(NB - 1) % 4]).wait()
        plsc.subcore_barrier()
        pltpu.sync_copy(agg_sh.at[pl.ds(r0, RPT)],
                        agg_hbm.at[cid, pl.ds(r0, RPT)])
        pltpu.sync_copy(deg_sh.at[pl.ds(r0, RPT)],
                        deg_hbm.at[cid, pl.ds(r0, RPT)])

    f = pl.kernel(
        body,
        out_type=(jax.ShapeDtypeStruct((NC, NP, H), jnp.float32),
                  jax.ShapeDtypeStruct((NC, NP, DW), jnp.float32)),
        mesh=mesh,
        scratch_types=[
            pltpu.VMEM((NB, B), jnp.int32),
            pltpu.VMEM((NB, B), jnp.int32),
            pltpu.VMEM((B, H), jnp.float32),
            pltpu.VMEM((B, H), jnp.float32),
            pltpu.VMEM((B, H), jnp.float32),
            pltpu.VMEM((B, H), jnp.float32),
            pltpu.VMEM((B, DW), jnp.float32),
            pltpu.VMEM_SHARED((NP, H), jnp.float32),
            pltpu.VMEM_SHARED((NP, DW), jnp.float32),
            pltpu.SemaphoreType.DMA,
            pltpu.SemaphoreType.DMA,
            pltpu.SemaphoreType.DMA,
            pltpu.SemaphoreType.DMA,
            pltpu.SemaphoreType.DMA,
            pltpu.SemaphoreType.DMA,
            pltpu.SemaphoreType.DMA,
            pltpu.SemaphoreType.DMA,
        ],
        compiler_params=pltpu.CompilerParams(use_tc_tiling_on_sc=False),
    )
    return f(ei4, y, z_agg, z_deg, ones)


def _epilogue(y, agg, deg, b1, W2, b2, W3, b3):
    N, H = y.shape
    BN = 2000
    G = N // BN

    def body(y_ref, agg_ref, deg_ref, b1_ref, W2_ref, b2_ref, W3_ref, b3_ref,
             o_ref, acc_ref):
        i = pl.program_id(0)

        @pl.when(i == 0)
        def _():
            acc_ref[...] = jnp.zeros_like(acc_ref)

        a = agg_ref[0] + agg_ref[1]
        d = jnp.maximum(deg_ref[0, :, 0:1] + deg_ref[1, :, 0:1], 1.0)
        h = jnp.maximum(y_ref[...] + a * (1.0 / d) + b1_ref[...], 0.0)
        h = jnp.maximum(
            jnp.dot(h, W2_ref[...], preferred_element_type=jnp.float32)
            + b2_ref[...], 0.0)
        acc_ref[...] += jnp.sum(h, axis=0, keepdims=True)

        @pl.when(i == G - 1)
        def _():
            m = acc_ref[...] / N
            o_ref[...] = jnp.tanh(
                jnp.dot(m, W3_ref[...], preferred_element_type=jnp.float32)
                + b3_ref[...])

    return pl.pallas_call(
        body,
        grid=(G,),
        in_specs=[
            pl.BlockSpec((BN, H), lambda i: (i, 0)),
            pl.BlockSpec((2, BN, H), lambda i: (0, i, 0)),
            pl.BlockSpec((2, BN, 16), lambda i: (0, i, 0)),
            pl.BlockSpec(b1.shape, lambda i: (0,)),
            pl.BlockSpec(W2.shape, lambda i: (0, 0)),
            pl.BlockSpec(b2.shape, lambda i: (0,)),
            pl.BlockSpec(W3.shape, lambda i: (0, 0)),
            pl.BlockSpec(b3.shape, lambda i: (0,)),
        ],
        out_specs=pl.BlockSpec((1, 1), lambda i: (0, 0)),
        out_shape=jax.ShapeDtypeStruct((1, 1), jnp.float32),
        scratch_shapes=[pltpu.VMEM((1, H), jnp.float32)],
    )(y, agg, deg, b1, W2, b2, W3, b3)


def kernel(x, edge_index, W1, b1, W2, b2, W3, b3):
    N, D = x.shape
    H = W1.shape[1]
    E = edge_index.shape[1]
    EPW = E // NW
    B = 125
    NB = EPW // B
    ei4 = edge_index.reshape(2, NW, NB, B)
    y = _project(x, W1)
    NP_ = ((N + 8 * NS - 1) // (8 * NS)) * (8 * NS)
    z_agg = jnp.zeros((NP_, H), jnp.float32)
    z_deg = jnp.zeros((NP_, 16), jnp.float32)
    ones = jnp.ones((B, 16), jnp.float32)
    agg, deg = _sc_aggregate(ei4, y, z_agg, z_deg, ones)
    out = _epilogue(y, agg, deg, b1, W2, b2, W3, b3)
    return out[0, 0]
